# baseline (speedup 1.0000x reference)
import functools

import jax
import jax.numpy as jnp
from jax import lax
from jax.experimental import pallas as pl
from jax.experimental.pallas import tpu as pltpu
from jax.experimental.pallas import tpu_sc as plsc

EMB = 128
L = 16
NP = 10240
E_TOTAL = 160000
NTILES = 16
NCORES = 2
EDGES_PER_TILE = E_TOTAL // NTILES
NODES_PER_TILE = NP // NTILES
EDGES_PER_WORKER = E_TOTAL // (NCORES * NTILES)
CHUNK = 80
ROUNDS = 3
REV_ROUNDS = 3

_f32 = jnp.float32


def _prop_body(hid_hbm, tid_hbm, topic_flat, pfr_hbm,
               hidx_v, tidx_v, msgs_f, msgs_r, ones_v, cur_f, cur_r,
               tmp_v, new_v, zero_v, invf_v, invr_v,
               acc_cf, acc_cr, acc_f, acc_r,
               sem_a, sem_b, sem_c, sem_d):
    c = lax.axis_index("c")
    s = lax.axis_index("s")
    ebase = s * EDGES_PER_TILE
    nbase = s * NODES_PER_TILE
    nsl = pl.ds(nbase, NODES_PER_TILE)

    pltpu.sync_copy(hid_hbm.at[pl.ds(ebase, EDGES_PER_TILE)], hidx_v)
    pltpu.sync_copy(tid_hbm.at[pl.ds(ebase, EDGES_PER_TILE)], tidx_v)

    zvec = jnp.zeros((L,), _f32)
    ovec = jnp.ones((L,), _f32)

    @pl.loop(0, NODES_PER_TILE // L)
    def _(i):
        zero_v[pl.ds(i * L, L)] = zvec

    @plsc.parallel_loop(0, EDGES_PER_TILE, step=L)
    def _(i):
        ones_v[pl.ds(i, L)] = ovec

    for acc in (acc_cf, acc_cr, acc_f, acc_r):
        pltpu.sync_copy(zero_v, acc.at[nsl])

    def gather(cur_v, sidx_v, out_v):
        @plsc.parallel_loop(0, EDGES_PER_TILE, step=L, unroll=8)
        def _(i):
            idx = sidx_v[pl.ds(i, L)]
            out_v[pl.ds(i, L)] = plsc.load_gather(cur_v, [idx])

    def node_slice(acc, inv_v, dst_slot):
        pltpu.sync_copy(acc.at[nsl], tmp_v)
        pltpu.sync_copy(zero_v, acc.at[nsl])

        @pl.loop(0, NODES_PER_TILE // L)
        def _(i):
            sl = pl.ds(i * L, L)
            new_v[sl] = tmp_v[sl] * inv_v[sl]

        pltpu.sync_copy(new_v, pfr_hbm.at[pl.ds(dst_slot + nbase,
                                                NODES_PER_TILE)])

    def inv_slice(acc, inv_v):
        pltpu.sync_copy(acc.at[nsl], tmp_v)

        @pl.loop(0, NODES_PER_TILE // L)
        def _(i):
            sl = pl.ds(i * L, L)
            inv_v[sl] = 1.0 / jnp.maximum(tmp_v[sl], 1.0)

    def slot(k):
        return k * NP

    pltpu.sync_copy(topic_flat.at[pl.ds(c * NP, NP)], cur_f)
    pltpu.sync_copy(cur_f, pfr_hbm.at[pl.ds(slot(c), NP)])
    plsc.subcore_barrier()

    d1 = pltpu.async_copy(ones_v, acc_cf.at[tidx_v], sem_a, add=True)
    d2 = pltpu.async_copy(ones_v, acc_cr.at[hidx_v], sem_b, add=True)
    gather(cur_f, hidx_v, msgs_f)
    d3 = pltpu.async_copy(msgs_f, acc_f.at[tidx_v], sem_c, add=True)
    gather(cur_f, tidx_v, msgs_r)
    d4 = pltpu.async_copy(msgs_r, acc_r.at[hidx_v], sem_d, add=True)
    d1.wait()
    d2.wait()
    d3.wait()
    d4.wait()
    plsc.subcore_barrier()

    inv_slice(acc_cf, invf_v)
    inv_slice(acc_cr, invr_v)
    node_slice(acc_f, invf_v, slot(2 + c))
    node_slice(acc_r, invr_v, slot(2 + 2 * ROUNDS + c))
    plsc.subcore_barrier()

    for r in range(1, ROUNDS):
        ca = pltpu.async_copy(pfr_hbm.at[pl.ds(slot(2 * r + c), NP)],
                              cur_f, sem_a)
        cb = pltpu.async_copy(
            pfr_hbm.at[pl.ds(slot(2 * (ROUNDS + r) + c), NP)], cur_r, sem_b)
        ca.wait()
        gather(cur_f, hidx_v, msgs_f)
        d3 = pltpu.async_copy(msgs_f, acc_f.at[tidx_v], sem_c, add=True)
        cb.wait()
        gather(cur_r, tidx_v, msgs_r)
        d4 = pltpu.async_copy(msgs_r, acc_r.at[hidx_v], sem_d, add=True)
        d3.wait()
        d4.wait()
        plsc.subcore_barrier()
        node_slice(acc_f, invf_v, slot(2 * (r + 1) + c))
        node_slice(acc_r, invr_v, slot(2 * (ROUNDS + r + 1) + c))
        plsc.subcore_barrier()


def _sc_mesh():
    return plsc.VectorSubcoreMesh(core_axis_name="c", subcore_axis_name="s",
                                  num_cores=NCORES, num_subcores=NTILES)


_SC_PARAMS = pltpu.CompilerParams(needs_layout_passes=False)


def _prop_call(h_id, t_id, topic_cc):
    mesh = _sc_mesh()
    nslots = 2 * (1 + ROUNDS + REV_ROUNDS)
    f = pl.kernel(
        _prop_body,
        out_type=jax.ShapeDtypeStruct((nslots * NP,), _f32),
        mesh=mesh,
        compiler_params=_SC_PARAMS,
        scratch_types=[
            pltpu.VMEM((EDGES_PER_TILE,), jnp.int32),
            pltpu.VMEM((EDGES_PER_TILE,), jnp.int32),
            pltpu.VMEM((EDGES_PER_TILE,), _f32),
            pltpu.VMEM((EDGES_PER_TILE,), _f32),
            pltpu.VMEM((EDGES_PER_TILE,), _f32),
            pltpu.VMEM((NP,), _f32),
            pltpu.VMEM((NP,), _f32),
            pltpu.VMEM((NODES_PER_TILE,), _f32),
            pltpu.VMEM((NODES_PER_TILE,), _f32),
            pltpu.VMEM((NODES_PER_TILE,), _f32),
            pltpu.VMEM((NODES_PER_TILE,), _f32),
            pltpu.VMEM((NODES_PER_TILE,), _f32),
            pltpu.VMEM_SHARED((NP,), _f32),
            pltpu.VMEM_SHARED((NP,), _f32),
            pltpu.VMEM_SHARED((NP,), _f32),
            pltpu.VMEM_SHARED((NP,), _f32),
            pltpu.SemaphoreType.DMA,
            pltpu.SemaphoreType.DMA,
            pltpu.SemaphoreType.DMA,
            pltpu.SemaphoreType.DMA,
        ],
    )
    return f(h_id, t_id, topic_cc.reshape(-1)).reshape(-1, NP)


def _dense_ab_body(x_ref, pfr_ref, wemb_ref, wpe_ref, a_ref, b_ref):
    ab = jnp.dot(x_ref[...], wemb_ref[...], preferred_element_type=_f32)
    ab = ab + jax.lax.dot_general(
        pfr_ref[...], wpe_ref[...], (((0,), (0,)), ((), ())),
        preferred_element_type=_f32)
    a_ref[...] = ab[:, :EMB]
    b_ref[...] = ab[:, EMB:]


def _dense_ab_call(h_e0, pfr, wemb2, wpe2):
    n = h_e0.shape[0]
    nslots = pfr.shape[0]
    bm = 1280
    grid = (n // bm,)
    return pl.pallas_call(
        _dense_ab_body,
        grid=grid,
        in_specs=[
            pl.BlockSpec((bm, EMB), lambda i: (i, 0)),
            pl.BlockSpec((nslots, bm), lambda i: (0, i)),
            pl.BlockSpec((EMB, 2 * EMB), lambda i: (0, 0)),
            pl.BlockSpec((nslots, 2 * EMB), lambda i: (0, 0)),
        ],
        out_specs=[
            pl.BlockSpec((bm, EMB), lambda i: (i, 0)),
            pl.BlockSpec((bm, EMB), lambda i: (i, 0)),
        ],
        out_shape=[jax.ShapeDtypeStruct((n, EMB), _f32),
                   jax.ShapeDtypeStruct((n, EMB), _f32)],
    )(h_e0, pfr, wemb2, wpe2)


def _dense_rm_body(rel_ref, wr_ref, q8_ref, wq_ref, b18_ref, rm_ref):
    qv = jnp.dot(q8_ref[...], wq_ref[...], preferred_element_type=_f32)
    rm = jnp.dot(rel_ref[...], wr_ref[...], preferred_element_type=_f32)
    rm_ref[...] = rm + qv[0:1, :] + b18_ref[0:1, :]


RM_REP = 16


def _dense_rm_call(rel, wr, q8, wq, b18):
    nr = rel.shape[0]
    return pl.pallas_call(
        _dense_rm_body,
        grid=(RM_REP,),
        in_specs=[
            pl.BlockSpec((nr, EMB), lambda j: (0, 0)),
            pl.BlockSpec((EMB, EMB), lambda j: (0, 0)),
            pl.BlockSpec((8, EMB), lambda j: (0, 0)),
            pl.BlockSpec((EMB, EMB), lambda j: (0, 0)),
            pl.BlockSpec((8, EMB), lambda j: (0, 0)),
        ],
        out_specs=pl.BlockSpec((nr, EMB), lambda j: (j, 0)),
        out_shape=jax.ShapeDtypeStruct((RM_REP * nr, EMB), _f32),
    )(rel, wr, q8, wq, b18)


def _edge_body(hid_hbm, rid_hbm, tid_hbm, a_hbm, b_hbm, rm_hbm, w2_hbm,
               b2_hbm, out_hbm, hidx_v, ridx_v, tidx_v,
               ba0, ba1, ba2, br0, br1, br2, bb0, bb1, bb2,
               out_v, w2_v, b2_v,
               sa0, sa1, sa2, sr0, sr1, sr2, sb0, sb1, sb2):
    c = lax.axis_index("c")
    s = lax.axis_index("s")
    w = s * NCORES + c
    ebase = w * EDGES_PER_WORKER

    pltpu.sync_copy(hid_hbm.at[pl.ds(ebase, EDGES_PER_WORKER)], hidx_v)
    pltpu.sync_copy(rid_hbm.at[pl.ds(ebase, EDGES_PER_WORKER)], ridx_v)
    pltpu.sync_copy(tid_hbm.at[pl.ds(ebase, EDGES_PER_WORKER)], tidx_v)
    pltpu.sync_copy(w2_hbm, w2_v)
    pltpu.sync_copy(b2_hbm, b2_v)

    rep_off = lax.iota(jnp.int32, L) * 512

    @plsc.parallel_loop(0, EDGES_PER_WORKER, step=L)
    def _(i):
        ridx_v[pl.ds(i, L)] = ridx_v[pl.ds(i, L)] + rep_off

    w2s = [w2_v[pl.ds(k * L, L)] for k in range(EMB // L)]
    b2vec = b2_v[...]
    lane = lax.iota(jnp.int32, L)
    bas = (ba0, ba1, ba2)
    brs = (br0, br1, br2)
    bbs = (bb0, bb1, bb2)
    sas = (sa0, sa1, sa2)
    srs = (sr0, sr1, sr2)
    sbs = (sb0, sb1, sb2)

    def g1(ch, q, n=CHUNK):
        return (a_hbm.at[hidx_v.at[pl.ds(ch * CHUNK, n)]], bas[q].at[:n],
                sas[q])

    def g2(ch, q, n=CHUNK):
        return (rm_hbm.at[ridx_v.at[pl.ds(ch * CHUNK, n)]], brs[q].at[:n],
                srs[q])

    def g3(ch, q, n=CHUNK):
        return (b_hbm.at[tidx_v.at[pl.ds(ch * CHUNK, n)]], bbs[q].at[:n],
                sbs[q])

    def issue(ch, q, n=CHUNK):
        pltpu.async_copy(*g1(ch, q, n))
        pltpu.async_copy(*g2(ch, q, n))
        pltpu.async_copy(*g3(ch, q, n))

    def wait(ch, q, n=CHUNK):
        pltpu.make_async_copy(*g1(ch, q, n)).wait()
        pltpu.make_async_copy(*g2(ch, q, n)).wait()
        pltpu.make_async_copy(*g3(ch, q, n)).wait()

    def compute(ch, q, ngroups=CHUNK // L):
        base = ch * CHUNK
        bufa, bufr, bufb = bas[q], brs[q], bbs[q]

        @plsc.parallel_loop(0, ngroups * L, step=L)
        def _(gbase):
            vec = jnp.zeros((L,), _f32)
            for j in range(L):
                e = gbase + j
                acc = b2vec
                for k in range(EMB // L):
                    ksl = pl.ds(k * L, L)
                    v = bufa[e, ksl] + bufr[e, ksl] + bufb[e, ksl]
                    acc = acc + jnp.maximum(v, 0.0) * w2s[k]
                vec = jnp.where(lane == j, jnp.sum(acc), vec)
            out_v[pl.ds(base + gbase, L)] = vec

    NFULL = EDGES_PER_WORKER // CHUNK
    TAIL = EDGES_PER_WORKER - NFULL * CHUNK

    issue(0, 0)
    issue(1, 1)

    @pl.loop(0, (NFULL + 2) // 3)
    def _(p3):
        for q in range(3):
            ch = p3 * 3 + q

            @pl.when(ch < NFULL)
            def _():
                wait(ch, q)

                @pl.when(ch + 2 < NFULL)
                def _():
                    issue(ch + 2, (q + 2) % 3)

                compute(ch, q)

    issue(NFULL, 0, TAIL)
    wait(NFULL, 0, TAIL)
    compute(NFULL, 0, (TAIL + L - 1) // L)

    pltpu.sync_copy(out_v.at[:EDGES_PER_WORKER],
                    out_hbm.at[pl.ds(ebase, EDGES_PER_WORKER)])


def _edge_call(h_id, r_id, t_id, a, b, rm, w2p, b2v):
    mesh = _sc_mesh()
    f = pl.kernel(
        _edge_body,
        out_type=jax.ShapeDtypeStruct((E_TOTAL,), _f32),
        mesh=mesh,
        compiler_params=_SC_PARAMS,
        scratch_types=(
            [pltpu.VMEM((EDGES_PER_WORKER,), jnp.int32)] * 3
            + [pltpu.VMEM((CHUNK, EMB), _f32)] * 9
            + [pltpu.VMEM((EDGES_PER_WORKER + L,), _f32),
               pltpu.VMEM((EMB,), _f32),
               pltpu.VMEM((L,), _f32)]
            + [pltpu.SemaphoreType.DMA] * 9
        ),
    )
    return f(h_id, r_id, t_id, a, b, rm, w2p, b2v)


def kernel(h_id_tensor, r_id_tensor, t_id_tensor, q_emb, entity_embs,
           num_non_text_entities, relation_embs, topic_one_hot, nte_emb,
           W1, b1, W2, b2):
    n_text = entity_embs.shape[0]
    n_nodes = topic_one_hot.shape[0]
    n_nontext = n_nodes - n_text

    h_id = h_id_tensor.astype(jnp.int32)
    r_id = r_id_tensor.astype(jnp.int32)
    t_id = t_id_tensor.astype(jnp.int32)

    topic_cc = jnp.zeros((NCORES, NP), _f32).at[:, :n_nodes].set(
        topic_one_hot.T.astype(_f32))

    pfr = _prop_call(h_id, t_id, topic_cc)

    h_e0 = jnp.concatenate(
        [entity_embs, jnp.broadcast_to(nte_emb[0][None, :], (n_nontext, EMB)),
         jnp.zeros((NP - n_nodes, EMB), _f32)],
        axis=0)

    wemb2 = jnp.concatenate([W1[128:256], W1[398:526]], axis=1)
    wpe2 = jnp.concatenate([W1[256:270], W1[526:540]], axis=1)

    a, b = _dense_ab_call(h_e0, pfr, wemb2, wpe2)
    q8 = jnp.broadcast_to(q_emb, (8, EMB))
    b18 = jnp.broadcast_to(b1[None, :], (8, EMB))
    rm = _dense_rm_call(relation_embs, W1[270:398], q8, W1[0:128], b18)

    w2p = W2[:, 0]
    b2v = jnp.zeros((L,), _f32).at[0].set(b2[0])

    out = _edge_call(h_id, r_id, t_id, a, b, rm, w2p, b2v)
    return out[:, None]


# speedup vs baseline: 1.5926x; 1.5926x over previous
import functools

import jax
import jax.numpy as jnp
from jax import lax
from jax.experimental import pallas as pl
from jax.experimental.pallas import tpu as pltpu
from jax.experimental.pallas import tpu_sc as plsc

EMB = 128
L = 16
NP = 10240
E_TOTAL = 160000
NTILES = 16
NCORES = 2
EDGES_PER_TILE = E_TOTAL // NTILES
NODES_PER_TILE = NP // NTILES
EDGES_PER_WORKER = E_TOTAL // (NCORES * NTILES)
CHUNK = 256
ROUNDS = 3
REV_ROUNDS = 3

_f32 = jnp.float32


def _prop_body(hid_hbm, tid_hbm, topic_flat, pfr_hbm,
               hidx_v, tidx_v, msgs_f, msgs_r, ones_v, cur_f, cur_r,
               tmp_v, new_v, zero_v, invf_v, invr_v,
               acc_cf, acc_cr, acc_f, acc_r,
               sem_a, sem_b, sem_c, sem_d):
    c = lax.axis_index("c")
    s = lax.axis_index("s")
    ebase = s * EDGES_PER_TILE
    nbase = s * NODES_PER_TILE
    nsl = pl.ds(nbase, NODES_PER_TILE)

    pltpu.sync_copy(hid_hbm.at[pl.ds(ebase, EDGES_PER_TILE)], hidx_v)
    pltpu.sync_copy(tid_hbm.at[pl.ds(ebase, EDGES_PER_TILE)], tidx_v)

    zvec = jnp.zeros((L,), _f32)
    ovec = jnp.ones((L,), _f32)

    @pl.loop(0, NODES_PER_TILE // L)
    def _(i):
        zero_v[pl.ds(i * L, L)] = zvec

    @plsc.parallel_loop(0, EDGES_PER_TILE, step=L)
    def _(i):
        ones_v[pl.ds(i, L)] = ovec

    for acc in (acc_cf, acc_cr, acc_f, acc_r):
        pltpu.sync_copy(zero_v, acc.at[nsl])

    def gather(cur_v, sidx_v, out_v):
        @plsc.parallel_loop(0, EDGES_PER_TILE, step=L, unroll=8)
        def _(i):
            idx = sidx_v[pl.ds(i, L)]
            out_v[pl.ds(i, L)] = plsc.load_gather(cur_v, [idx])

    def node_slice(acc, inv_v, dst_slot):
        pltpu.sync_copy(acc.at[nsl], tmp_v)
        pltpu.sync_copy(zero_v, acc.at[nsl])

        @pl.loop(0, NODES_PER_TILE // L)
        def _(i):
            sl = pl.ds(i * L, L)
            new_v[sl] = tmp_v[sl] * inv_v[sl]

        pltpu.sync_copy(new_v, pfr_hbm.at[pl.ds(dst_slot + nbase,
                                                NODES_PER_TILE)])

    def inv_slice(acc, inv_v):
        pltpu.sync_copy(acc.at[nsl], tmp_v)

        @pl.loop(0, NODES_PER_TILE // L)
        def _(i):
            sl = pl.ds(i * L, L)
            inv_v[sl] = 1.0 / jnp.maximum(tmp_v[sl], 1.0)

    def slot(k):
        return k * NP

    pltpu.sync_copy(topic_flat.at[pl.ds(c * NP, NP)], cur_f)
    pltpu.sync_copy(cur_f, pfr_hbm.at[pl.ds(slot(c), NP)])
    plsc.subcore_barrier()

    d1 = pltpu.async_copy(ones_v, acc_cf.at[tidx_v], sem_a, add=True)
    d2 = pltpu.async_copy(ones_v, acc_cr.at[hidx_v], sem_b, add=True)
    gather(cur_f, hidx_v, msgs_f)
    d3 = pltpu.async_copy(msgs_f, acc_f.at[tidx_v], sem_c, add=True)
    gather(cur_f, tidx_v, msgs_r)
    d4 = pltpu.async_copy(msgs_r, acc_r.at[hidx_v], sem_d, add=True)
    d1.wait()
    d2.wait()
    d3.wait()
    d4.wait()
    plsc.subcore_barrier()

    inv_slice(acc_cf, invf_v)
    inv_slice(acc_cr, invr_v)
    node_slice(acc_f, invf_v, slot(2 + c))
    node_slice(acc_r, invr_v, slot(2 + 2 * ROUNDS + c))
    plsc.subcore_barrier()

    for r in range(1, ROUNDS):
        ca = pltpu.async_copy(pfr_hbm.at[pl.ds(slot(2 * r + c), NP)],
                              cur_f, sem_a)
        cb = pltpu.async_copy(
            pfr_hbm.at[pl.ds(slot(2 * (ROUNDS + r) + c), NP)], cur_r, sem_b)
        ca.wait()
        gather(cur_f, hidx_v, msgs_f)
        d3 = pltpu.async_copy(msgs_f, acc_f.at[tidx_v], sem_c, add=True)
        cb.wait()
        gather(cur_r, tidx_v, msgs_r)
        d4 = pltpu.async_copy(msgs_r, acc_r.at[hidx_v], sem_d, add=True)
        d3.wait()
        d4.wait()
        plsc.subcore_barrier()
        node_slice(acc_f, invf_v, slot(2 * (r + 1) + c))
        node_slice(acc_r, invr_v, slot(2 * (ROUNDS + r + 1) + c))
        plsc.subcore_barrier()


def _sc_mesh():
    return plsc.VectorSubcoreMesh(core_axis_name="c", subcore_axis_name="s",
                                  num_cores=NCORES, num_subcores=NTILES)


_SC_PARAMS = pltpu.CompilerParams(needs_layout_passes=False)


def _prop_call(h_id, t_id, topic_cc):
    mesh = _sc_mesh()
    nslots = 2 * (1 + ROUNDS + REV_ROUNDS)
    f = pl.kernel(
        _prop_body,
        out_type=jax.ShapeDtypeStruct((nslots * NP,), _f32),
        mesh=mesh,
        compiler_params=_SC_PARAMS,
        scratch_types=[
            pltpu.VMEM((EDGES_PER_TILE,), jnp.int32),
            pltpu.VMEM((EDGES_PER_TILE,), jnp.int32),
            pltpu.VMEM((EDGES_PER_TILE,), _f32),
            pltpu.VMEM((EDGES_PER_TILE,), _f32),
            pltpu.VMEM((EDGES_PER_TILE,), _f32),
            pltpu.VMEM((NP,), _f32),
            pltpu.VMEM((NP,), _f32),
            pltpu.VMEM((NODES_PER_TILE,), _f32),
            pltpu.VMEM((NODES_PER_TILE,), _f32),
            pltpu.VMEM((NODES_PER_TILE,), _f32),
            pltpu.VMEM((NODES_PER_TILE,), _f32),
            pltpu.VMEM((NODES_PER_TILE,), _f32),
            pltpu.VMEM_SHARED((NP,), _f32),
            pltpu.VMEM_SHARED((NP,), _f32),
            pltpu.VMEM_SHARED((NP,), _f32),
            pltpu.VMEM_SHARED((NP,), _f32),
            pltpu.SemaphoreType.DMA,
            pltpu.SemaphoreType.DMA,
            pltpu.SemaphoreType.DMA,
            pltpu.SemaphoreType.DMA,
        ],
    )
    return f(h_id, t_id, topic_cc.reshape(-1)).reshape(-1, NP)


def _dense_ab_body(x_ref, pfr_ref, wemb_ref, wpe_ref, a_ref, b_ref):
    ab = jnp.dot(x_ref[...], wemb_ref[...], preferred_element_type=_f32)
    ab = ab + jax.lax.dot_general(
        pfr_ref[...], wpe_ref[...], (((0,), (0,)), ((), ())),
        preferred_element_type=_f32)
    a_ref[...] = ab[:, :EMB]
    b_ref[...] = ab[:, EMB:]


def _dense_ab_call(h_e0, pfr, wemb2, wpe2):
    n = h_e0.shape[0]
    nslots = pfr.shape[0]
    bm = 1280
    grid = (n // bm,)
    return pl.pallas_call(
        _dense_ab_body,
        grid=grid,
        in_specs=[
            pl.BlockSpec((bm, EMB), lambda i: (i, 0)),
            pl.BlockSpec((nslots, bm), lambda i: (0, i)),
            pl.BlockSpec((EMB, 2 * EMB), lambda i: (0, 0)),
            pl.BlockSpec((nslots, 2 * EMB), lambda i: (0, 0)),
        ],
        out_specs=[
            pl.BlockSpec((bm, EMB), lambda i: (i, 0)),
            pl.BlockSpec((bm, EMB), lambda i: (i, 0)),
        ],
        out_shape=[jax.ShapeDtypeStruct((n, EMB), _f32),
                   jax.ShapeDtypeStruct((n, EMB), _f32)],
    )(h_e0, pfr, wemb2, wpe2)


def _dense_rm_body(rel_ref, wr_ref, q8_ref, wq_ref, b18_ref, rm_ref):
    qv = jnp.dot(q8_ref[...], wq_ref[...], preferred_element_type=_f32)
    rm = jnp.dot(rel_ref[...], wr_ref[...], preferred_element_type=_f32)
    rm_ref[...] = rm + qv[0:1, :] + b18_ref[0:1, :]


RM_REP = 16


def _dense_rm_call(rel, wr, q8, wq, b18):
    nr = rel.shape[0]
    return pl.pallas_call(
        _dense_rm_body,
        grid=(RM_REP,),
        in_specs=[
            pl.BlockSpec((nr, EMB), lambda j: (0, 0)),
            pl.BlockSpec((EMB, EMB), lambda j: (0, 0)),
            pl.BlockSpec((8, EMB), lambda j: (0, 0)),
            pl.BlockSpec((EMB, EMB), lambda j: (0, 0)),
            pl.BlockSpec((8, EMB), lambda j: (0, 0)),
        ],
        out_specs=pl.BlockSpec((nr, EMB), lambda j: (j, 0)),
        out_shape=jax.ShapeDtypeStruct((RM_REP * nr, EMB), _f32),
    )(rel, wr, q8, wq, b18)


def _edge_body(hid_hbm, rid_hbm, tid_hbm, a_hbm, b_hbm, rm_hbm, w2_hbm,
               b2_hbm, out_hbm, hidx_v, ridx_v, tidx_v,
               buf0, buf1, buf2, out_v, w2_v, b2_v,
               sem1_0, sem1_1, sem1_2, sem23_0, sem23_1, sem23_2):
    c = lax.axis_index("c")
    s = lax.axis_index("s")
    w = s * NCORES + c
    ebase = w * EDGES_PER_WORKER

    pltpu.sync_copy(hid_hbm.at[pl.ds(ebase, EDGES_PER_WORKER)], hidx_v)
    pltpu.sync_copy(rid_hbm.at[pl.ds(ebase, EDGES_PER_WORKER)], ridx_v)
    pltpu.sync_copy(tid_hbm.at[pl.ds(ebase, EDGES_PER_WORKER)], tidx_v)

    rep_off = lax.iota(jnp.int32, L) * 512

    @plsc.parallel_loop(0, EDGES_PER_WORKER, step=L)
    def _(i):
        ridx_v[pl.ds(i, L)] = ridx_v[pl.ds(i, L)] + rep_off
    pltpu.sync_copy(w2_hbm, w2_v)
    pltpu.sync_copy(b2_hbm, b2_v)

    w2s = [w2_v[pl.ds(k * L, L)] for k in range(EMB // L)]
    b2vec = b2_v[...]
    lane = lax.iota(jnp.int32, L)
    bufs = (buf0, buf1, buf2)
    sem1 = (sem1_0, sem1_1, sem1_2)
    sem23 = (sem23_0, sem23_1, sem23_2)

    def g1(ch, q, n=CHUNK):
        return (a_hbm.at[hidx_v.at[pl.ds(ch * CHUNK, n)]],
                bufs[q].at[:n], sem1[q])

    def g2(ch, q, n=CHUNK):
        return (rm_hbm.at[ridx_v.at[pl.ds(ch * CHUNK, n)]],
                bufs[q].at[:n], sem23[q])

    def g3(ch, q, n=CHUNK):
        return (b_hbm.at[tidx_v.at[pl.ds(ch * CHUNK, n)]],
                bufs[q].at[:n], sem23[q])

    def compute(ch, q, ngroups=CHUNK // L):
        base = ch * CHUNK
        buf = bufs[q]

        @plsc.parallel_loop(0, ngroups * L, step=L)
        def _(gbase):
            vec = jnp.zeros((L,), _f32)
            for j in range(L):
                e = gbase + j
                acc = b2vec
                for k in range(EMB // L):
                    v = buf[e, pl.ds(k * L, L)]
                    acc = acc + jnp.maximum(v, 0.0) * w2s[k]
                vec = jnp.where(lane == j, jnp.sum(acc), vec)
            out_v[pl.ds(base + gbase, L)] = vec

    NFULL = EDGES_PER_WORKER // CHUNK
    TAIL = EDGES_PER_WORKER - NFULL * CHUNK

    pltpu.async_copy(*g1(0, 0))
    pltpu.async_copy(*g1(1, 1))
    pltpu.make_async_copy(*g1(0, 0)).wait()
    pltpu.async_copy(*g2(0, 0), add=True)
    pltpu.async_copy(*g3(0, 0), add=True)

    @pl.loop(0, (NFULL + 2) // 3)
    def _(p3):
        for q in range(3):
            ch = p3 * 3 + q

            @pl.when(ch < NFULL)
            def _():
                pltpu.make_async_copy(*g2(ch, q)).wait()
                pltpu.make_async_copy(*g3(ch, q)).wait()

                @pl.when(ch + 1 < NFULL)
                def _():
                    q1 = (q + 1) % 3
                    pltpu.make_async_copy(*g1(ch + 1, q1)).wait()
                    pltpu.async_copy(*g2(ch + 1, q1), add=True)
                    pltpu.async_copy(*g3(ch + 1, q1), add=True)

                @pl.when(ch + 2 < NFULL)
                def _():
                    pltpu.async_copy(*g1(ch + 2, (q + 2) % 3))

                compute(ch, q)

    pltpu.async_copy(*g1(NFULL, 0, TAIL))
    pltpu.make_async_copy(*g1(NFULL, 0, TAIL)).wait()
    pltpu.async_copy(*g2(NFULL, 0, TAIL), add=True)
    pltpu.async_copy(*g3(NFULL, 0, TAIL), add=True)
    pltpu.make_async_copy(*g2(NFULL, 0, TAIL)).wait()
    pltpu.make_async_copy(*g3(NFULL, 0, TAIL)).wait()
    compute(NFULL, 0, (TAIL + L - 1) // L)

    pltpu.sync_copy(out_v.at[:EDGES_PER_WORKER],
                    out_hbm.at[pl.ds(ebase, EDGES_PER_WORKER)])


def _edge_call(h_id, r_id, t_id, a, b, rm, w2p, b2v):
    mesh = _sc_mesh()
    f = pl.kernel(
        _edge_body,
        out_type=jax.ShapeDtypeStruct((E_TOTAL,), _f32),
        mesh=mesh,
        compiler_params=_SC_PARAMS,
        scratch_types=[
            pltpu.VMEM((EDGES_PER_WORKER,), jnp.int32),
            pltpu.VMEM((EDGES_PER_WORKER,), jnp.int32),
            pltpu.VMEM((EDGES_PER_WORKER,), jnp.int32),
            pltpu.VMEM((CHUNK, EMB), _f32),
            pltpu.VMEM((CHUNK, EMB), _f32),
            pltpu.VMEM((CHUNK, EMB), _f32),
            pltpu.VMEM((EDGES_PER_WORKER + L,), _f32),
            pltpu.VMEM((EMB,), _f32),
            pltpu.VMEM((L,), _f32),
            pltpu.SemaphoreType.DMA,
            pltpu.SemaphoreType.DMA,
            pltpu.SemaphoreType.DMA,
            pltpu.SemaphoreType.DMA,
            pltpu.SemaphoreType.DMA,
            pltpu.SemaphoreType.DMA,
        ],
    )
    return f(h_id, r_id, t_id, a, b, rm, w2p, b2v)


def kernel(h_id_tensor, r_id_tensor, t_id_tensor, q_emb, entity_embs,
           num_non_text_entities, relation_embs, topic_one_hot, nte_emb,
           W1, b1, W2, b2):
    n_text = entity_embs.shape[0]
    n_nodes = topic_one_hot.shape[0]
    n_nontext = n_nodes - n_text

    h_id = h_id_tensor.astype(jnp.int32)
    r_id = r_id_tensor.astype(jnp.int32)
    t_id = t_id_tensor.astype(jnp.int32)

    topic_cc = jnp.zeros((NCORES, NP), _f32).at[:, :n_nodes].set(
        topic_one_hot.T.astype(_f32))

    pfr = _prop_call(h_id, t_id, topic_cc)

    h_e0 = jnp.concatenate(
        [entity_embs, jnp.broadcast_to(nte_emb[0][None, :], (n_nontext, EMB)),
         jnp.zeros((NP - n_nodes, EMB), _f32)],
        axis=0)

    wemb2 = jnp.concatenate([W1[128:256], W1[398:526]], axis=1)
    wpe2 = jnp.concatenate([W1[256:270], W1[526:540]], axis=1)

    a, b = _dense_ab_call(h_e0, pfr, wemb2, wpe2)
    q8 = jnp.broadcast_to(q_emb, (8, EMB))
    b18 = jnp.broadcast_to(b1[None, :], (8, EMB))
    rm = _dense_rm_call(relation_embs, W1[270:398], q8, W1[0:128], b18)

    w2p = W2[:, 0]
    b2v = jnp.zeros((L,), _f32).at[0].set(b2[0])

    out = _edge_call(h_id, r_id, t_id, a, b, rm, w2p, b2v)
    return out[:, None]
